# baseline (speedup 1.0000x reference)
import functools

import jax
import jax.numpy as jnp
from jax.experimental import pallas as pl
from jax.experimental.pallas import tpu as pltpu


def _dense_fused_kernel(x_ref, s1_ref, b1_ref, w1_ref, b2_ref, w2r_ref,
                        o_ref, *, H, W, Cin, C1, Cout):
    HW = H * W
    x4 = x_ref[0]
    o_ref[0, :Cin] = x4

    x2 = x4.reshape(Cin, HW)

    h = jnp.maximum(x2 * s1_ref[...] + b1_ref[...], 0.0)

    t = jax.lax.dot_general(w1_ref[...], h, (((0,), (0,)), ((), ())),
                            preferred_element_type=jnp.float32)
    t = jnp.maximum(t + b2_ref[...], 0.0)

    col = jax.lax.broadcasted_iota(jnp.int32, (C1, HW), 1) % W
    t_l = jnp.where(col >= 1, pltpu.roll(t, 1, axis=1), 0.0)
    t_r = jnp.where(col <= W - 2, pltpu.roll(t, HW - 1, axis=1), 0.0)
    cat = jnp.concatenate([t_l, t, t_r], axis=0)

    p = jax.lax.dot_general(w2r_ref[...], cat, (((1,), (0,)), ((), ())),
                            preferred_element_type=jnp.float32)

    lane = jax.lax.broadcasted_iota(jnp.int32, (Cout, HW), 1)
    y = p[Cout:2 * Cout]
    y = y + jnp.where(lane >= W, pltpu.roll(p[:Cout], W, axis=1), 0.0)
    y = y + jnp.where(lane < HW - W,
                      pltpu.roll(p[2 * Cout:], HW - W, axis=1), 0.0)

    o_ref[0, Cin:] = y.reshape(Cout, H, W)


def kernel(x, s1, b1, w1_eff, b2, w2):
    N, Cin, H, W = x.shape
    C1 = w1_eff.shape[1]
    Cout = w2.shape[1]

    s1c = s1.reshape(Cin, 1)
    b1c = b1.reshape(Cin, 1)
    b2c = b2.reshape(C1, 1)
    w2r = w2.reshape(3, 3 * C1, Cout).transpose(0, 2, 1).reshape(3 * Cout,
                                                                 3 * C1)

    out = pl.pallas_call(
        functools.partial(_dense_fused_kernel, H=H, W=W, Cin=Cin, C1=C1,
                          Cout=Cout),
        out_shape=jax.ShapeDtypeStruct((N, Cin + Cout, H, W), jnp.float32),
        grid=(N,),
        in_specs=[
            pl.BlockSpec((1, Cin, H, W), lambda n: (n, 0, 0, 0)),
            pl.BlockSpec((Cin, 1), lambda n: (0, 0)),
            pl.BlockSpec((Cin, 1), lambda n: (0, 0)),
            pl.BlockSpec((Cin, C1), lambda n: (0, 0)),
            pl.BlockSpec((C1, 1), lambda n: (0, 0)),
            pl.BlockSpec((3 * Cout, 3 * C1), lambda n: (0, 0)),
        ],
        out_specs=pl.BlockSpec((1, Cin + Cout, H, W), lambda n: (n, 0, 0, 0)),
        compiler_params=pltpu.CompilerParams(
            dimension_semantics=("parallel",)),
    )(x, s1c, b1c, w1_eff, b2c, w2r)

    return out


# speedup vs baseline: 7.7903x; 7.7903x over previous
import functools

import jax
import jax.numpy as jnp
from jax.experimental import pallas as pl
from jax.experimental.pallas import tpu as pltpu


def _dense_fused_kernel(x_ref, s1_ref, b1_ref, w1_ref, b2_ref, w2_ref,
                        o_ref, *, H, W, B, Cin, C1, Cout):
    HW = H * W
    M = HW * B
    SH = W * B

    o_ref[:, :, :Cin] = x_ref[...]

    x2 = x_ref[...].reshape(M, Cin)

    h = jnp.maximum(x2 * s1_ref[...] + b1_ref[...], 0.0)

    t = jnp.dot(h, w1_ref[...], preferred_element_type=jnp.float32)
    t = jnp.maximum(t + b2_ref[...], 0.0)

    jj = (jax.lax.broadcasted_iota(jnp.int32, (M, C1), 0) // B) % W
    zB = jnp.zeros((B, C1), jnp.float32)
    t_l = jnp.where(jj >= 1, jnp.concatenate([zB, t[:M - B]], axis=0), 0.0)
    t_r = jnp.where(jj <= W - 2, jnp.concatenate([t[B:], zB], axis=0), 0.0)

    def pa(a):
        acc = jnp.dot(t_l, w2_ref[3 * a * C1:(3 * a + 1) * C1],
                      preferred_element_type=jnp.float32)
        acc = acc + jnp.dot(t, w2_ref[(3 * a + 1) * C1:(3 * a + 2) * C1],
                            preferred_element_type=jnp.float32)
        return acc + jnp.dot(t_r, w2_ref[(3 * a + 2) * C1:(3 * a + 3) * C1],
                             preferred_element_type=jnp.float32)

    p0, p1, p2 = pa(0), pa(1), pa(2)

    zSH = jnp.zeros((SH, Cout), jnp.float32)
    y = (p1 + jnp.concatenate([zSH, p0[:M - SH]], axis=0)
         + jnp.concatenate([p2[SH:], zSH], axis=0))

    o_ref[:, :, Cin:] = y.reshape(HW, B, Cout)


def kernel(x, s1, b1, w1_eff, b2, w2):
    N, Cin, H, W = x.shape
    C1 = w1_eff.shape[1]
    Cout = w2.shape[1]
    HW = H * W
    B = 8 if N % 8 == 0 else N
    G = N // B
    NC = 1
    GK = G // NC

    xt = jnp.transpose(x, (2, 3, 0, 1)).reshape(HW, N, Cin)

    out_t = pl.pallas_call(
        functools.partial(_dense_fused_kernel, H=H, W=W, B=B, Cin=Cin,
                          C1=C1, Cout=Cout),
        out_shape=jax.ShapeDtypeStruct((HW, N, Cin + Cout), jnp.float32),
        grid=(NC, GK),
        in_specs=[
            pl.BlockSpec((HW, B, Cin), lambda c, k: (0, c * GK + k, 0)),
            pl.BlockSpec((1, Cin), lambda c, k: (0, 0)),
            pl.BlockSpec((1, Cin), lambda c, k: (0, 0)),
            pl.BlockSpec((Cin, C1), lambda c, k: (0, 0)),
            pl.BlockSpec((1, C1), lambda c, k: (0, 0)),
            pl.BlockSpec((9 * C1, Cout), lambda c, k: (0, 0)),
        ],
        out_specs=pl.BlockSpec((HW, B, Cin + Cout),
                               lambda c, k: (0, c * GK + k, 0)),
        compiler_params=pltpu.CompilerParams(
            dimension_semantics=("arbitrary", "arbitrary"),
            vmem_limit_bytes=100 * 1024 * 1024),
    )(xt, s1, b1, w1_eff, b2, w2)

    return jnp.transpose(out_t.reshape(H, W, N, Cin + Cout), (2, 3, 0, 1))


# speedup vs baseline: 7.9486x; 1.0203x over previous
import functools

import jax
import jax.numpy as jnp
from jax.experimental import pallas as pl
from jax.experimental.pallas import tpu as pltpu


def _dense_fused_kernel(x_ref, s1_ref, b1_ref, w1_ref, b2_ref, w2t_ref,
                        o_ref, *, H, W, B, Cin, C1, Cout):
    HW = H * W
    M = HW * B
    SH = W * B

    o_ref[:, :, :Cin] = x_ref[...]

    x2 = x_ref[...].reshape(M, Cin)

    h = jnp.maximum(x2 * s1_ref[...] + b1_ref[...], 0.0)

    t = jnp.dot(h, w1_ref[...], preferred_element_type=jnp.float32)
    t = jnp.maximum(t + b2_ref[...], 0.0)

    p_all = jnp.dot(t, w2t_ref[...], preferred_element_type=jnp.float32)

    def tap(a, b):
        k = (3 * a + b) * Cout
        return p_all[:, k:k + Cout]

    def shift(p, k):
        if k == 0:
            return p
        z = jnp.zeros((abs(k), Cout), jnp.float32)
        if k > 0:
            return jnp.concatenate([p[k:], z], axis=0)
        return jnp.concatenate([z, p[:M + k]], axis=0)

    def qb(b):
        return (tap(1, b) + shift(tap(0, b), -SH) + shift(tap(2, b), SH))

    jj = (jax.lax.broadcasted_iota(jnp.int32, (M, Cout), 0) // B) % W
    y = (qb(1)
         + jnp.where(jj >= 1, shift(qb(0), -B), 0.0)
         + jnp.where(jj <= W - 2, shift(qb(2), B), 0.0))

    o_ref[:, :, Cin:] = y.reshape(HW, B, Cout)


def kernel(x, s1, b1, w1_eff, b2, w2):
    N, Cin, H, W = x.shape
    C1 = w1_eff.shape[1]
    Cout = w2.shape[1]
    HW = H * W
    B = 8 if N % 8 == 0 else N
    G = N // B
    NC = 1
    GK = G // NC

    w2t = jnp.transpose(w2.reshape(9, C1, Cout), (1, 0, 2)).reshape(C1, 9 * Cout)

    xt = jnp.transpose(x, (2, 3, 0, 1)).reshape(HW, N, Cin)

    out_t = pl.pallas_call(
        functools.partial(_dense_fused_kernel, H=H, W=W, B=B, Cin=Cin,
                          C1=C1, Cout=Cout),
        out_shape=jax.ShapeDtypeStruct((HW, N, Cin + Cout), jnp.float32),
        grid=(NC, GK),
        in_specs=[
            pl.BlockSpec((HW, B, Cin), lambda c, k: (0, c * GK + k, 0)),
            pl.BlockSpec((1, Cin), lambda c, k: (0, 0)),
            pl.BlockSpec((1, Cin), lambda c, k: (0, 0)),
            pl.BlockSpec((Cin, C1), lambda c, k: (0, 0)),
            pl.BlockSpec((1, C1), lambda c, k: (0, 0)),
            pl.BlockSpec((C1, 9 * Cout), lambda c, k: (0, 0)),
        ],
        out_specs=pl.BlockSpec((HW, B, Cin + Cout),
                               lambda c, k: (0, c * GK + k, 0)),
        compiler_params=pltpu.CompilerParams(
            dimension_semantics=("arbitrary", "arbitrary"),
            vmem_limit_bytes=100 * 1024 * 1024),
    )(xt, s1, b1, w1_eff, b2, w2t)

    return jnp.transpose(out_t.reshape(H, W, N, Cin + Cout), (2, 3, 0, 1))


# speedup vs baseline: 8.2662x; 1.0400x over previous
import functools

import jax
import jax.numpy as jnp
from jax.experimental import pallas as pl
from jax.experimental.pallas import tpu as pltpu


def _dense_fused_kernel(x_ref, s1_ref, b1_ref, w1_ref, b2_ref, w2t_ref,
                        o_ref, copy_sem, *, H, W, B, Cin, C1, Cout):
    HW = H * W
    M = HW * B
    SH = W * B

    cp = pltpu.make_async_copy(x_ref, o_ref.at[:, :, :Cin], copy_sem)
    cp.start()

    x2 = x_ref[...].reshape(M, Cin)

    h = jnp.maximum(x2 * s1_ref[...] + b1_ref[...], 0.0)

    t = jnp.dot(h, w1_ref[...], preferred_element_type=jnp.float32)
    t = jnp.maximum(t + b2_ref[...], 0.0)

    p_all = jnp.dot(t, w2t_ref[...], preferred_element_type=jnp.float32)

    WB = W * B
    p3 = p_all.reshape(H, WB, 9 * Cout)

    def tap(a, b):
        k = (3 * a + b) * Cout
        return p3[:, :, k:k + Cout]

    def colshift(p, k):
        z = jnp.zeros((H, abs(k), Cout), jnp.float32)
        if k > 0:
            return jnp.concatenate([p[:, k:], z], axis=1)
        return jnp.concatenate([z, p[:, :WB + k]], axis=1)

    def rowshift(p, k):
        z = jnp.zeros((abs(k), WB, Cout), jnp.float32)
        if k > 0:
            return jnp.concatenate([p[k:], z], axis=0)
        return jnp.concatenate([z, p[:H + k]], axis=0)

    def qb(b):
        return (tap(1, b) + rowshift(tap(0, b), -1) + rowshift(tap(2, b), 1))

    y = qb(1) + colshift(qb(0), -B) + colshift(qb(2), B)

    o_ref[:, :, Cin:] = y.reshape(HW, B, Cout)
    cp.wait()


def kernel(x, s1, b1, w1_eff, b2, w2):
    N, Cin, H, W = x.shape
    C1 = w1_eff.shape[1]
    Cout = w2.shape[1]
    HW = H * W
    B = 8 if N % 8 == 0 else N
    G = N // B
    NC = 1
    GK = G // NC

    w2t = jnp.transpose(w2.reshape(9, C1, Cout), (1, 0, 2)).reshape(C1, 9 * Cout)

    xt = jnp.transpose(x, (2, 3, 0, 1)).reshape(HW, N, Cin)

    out_t = pl.pallas_call(
        functools.partial(_dense_fused_kernel, H=H, W=W, B=B, Cin=Cin,
                          C1=C1, Cout=Cout),
        out_shape=jax.ShapeDtypeStruct((HW, N, Cin + Cout), jnp.float32),
        grid=(NC, GK),
        in_specs=[
            pl.BlockSpec((HW, B, Cin), lambda c, k: (0, c * GK + k, 0)),
            pl.BlockSpec((1, Cin), lambda c, k: (0, 0)),
            pl.BlockSpec((1, Cin), lambda c, k: (0, 0)),
            pl.BlockSpec((Cin, C1), lambda c, k: (0, 0)),
            pl.BlockSpec((1, C1), lambda c, k: (0, 0)),
            pl.BlockSpec((C1, 9 * Cout), lambda c, k: (0, 0)),
        ],
        out_specs=pl.BlockSpec((HW, B, Cin + Cout),
                               lambda c, k: (0, c * GK + k, 0)),
        scratch_shapes=[pltpu.SemaphoreType.DMA],
        compiler_params=pltpu.CompilerParams(
            dimension_semantics=("arbitrary", "arbitrary"),
            vmem_limit_bytes=100 * 1024 * 1024),
    )(xt, s1, b1, w1_eff, b2, w2t)

    return jnp.transpose(out_t.reshape(H, W, N, Cin + Cout), (2, 3, 0, 1))


# speedup vs baseline: 10.3906x; 1.2570x over previous
import functools

import jax
import jax.numpy as jnp
from jax.experimental import pallas as pl
from jax.experimental.pallas import tpu as pltpu


def _dense_fused_kernel(x_ref, s1_ref, b1_ref, w1_ref, b2_ref, w2g_ref,
                        o_ref, copy_sem, *, H, W, B, Cin, C1, Cout):
    HW = H * W
    M = HW * B
    SH = W * B

    cp = pltpu.make_async_copy(x_ref, o_ref.at[:, :, :Cin], copy_sem)
    cp.start()

    x2 = x_ref[...].reshape(M, Cin)

    h = jnp.maximum(x2 * s1_ref[...] + b1_ref[...], 0.0)

    t = jnp.dot(h, w1_ref[...], preferred_element_type=jnp.float32)
    t = jnp.maximum(t + b2_ref[...], 0.0)

    WB = W * B

    def colshift(p, k):
        z = jnp.zeros((H, abs(k), p.shape[2]), jnp.float32)
        if k > 0:
            return jnp.concatenate([p[:, k:], z], axis=1)
        return jnp.concatenate([z, p[:, :WB + k]], axis=1)

    def rowshift(p, k):
        z = jnp.zeros((abs(k), WB, p.shape[2]), jnp.float32)
        if k > 0:
            return jnp.concatenate([p[k:], z], axis=0)
        return jnp.concatenate([z, p[:H + k]], axis=0)

    t3 = t.reshape(H, WB, C1)
    cat = jnp.concatenate([colshift(t3, -B), t3, colshift(t3, B)],
                          axis=2).reshape(M, 3 * C1)
    p = jnp.dot(cat, w2g_ref[...], preferred_element_type=jnp.float32)
    p3 = p.reshape(H, WB, 3 * Cout)

    y = (p3[:, :, Cout:2 * Cout]
         + rowshift(p3[:, :, :Cout], -1)
         + rowshift(p3[:, :, 2 * Cout:], 1))

    o_ref[:, :, Cin:] = y.reshape(HW, B, Cout)
    cp.wait()


def kernel(x, s1, b1, w1_eff, b2, w2):
    N, Cin, H, W = x.shape
    C1 = w1_eff.shape[1]
    Cout = w2.shape[1]
    HW = H * W
    B = 8 if N % 8 == 0 else N
    G = N // B
    NC = 1
    GK = G // NC

    w2g = jnp.transpose(w2.reshape(3, 3, C1, Cout),
                        (1, 2, 0, 3)).reshape(3 * C1, 3 * Cout)

    xt = jnp.transpose(x, (2, 3, 0, 1)).reshape(HW, N, Cin)

    out_t = pl.pallas_call(
        functools.partial(_dense_fused_kernel, H=H, W=W, B=B, Cin=Cin,
                          C1=C1, Cout=Cout),
        out_shape=jax.ShapeDtypeStruct((HW, N, Cin + Cout), jnp.float32),
        grid=(NC, GK),
        in_specs=[
            pl.BlockSpec((HW, B, Cin), lambda c, k: (0, c * GK + k, 0)),
            pl.BlockSpec((1, Cin), lambda c, k: (0, 0)),
            pl.BlockSpec((1, Cin), lambda c, k: (0, 0)),
            pl.BlockSpec((Cin, C1), lambda c, k: (0, 0)),
            pl.BlockSpec((1, C1), lambda c, k: (0, 0)),
            pl.BlockSpec((3 * C1, 3 * Cout), lambda c, k: (0, 0)),
        ],
        out_specs=pl.BlockSpec((HW, B, Cin + Cout),
                               lambda c, k: (0, c * GK + k, 0)),
        scratch_shapes=[pltpu.SemaphoreType.DMA],
        compiler_params=pltpu.CompilerParams(
            dimension_semantics=("arbitrary", "arbitrary"),
            vmem_limit_bytes=100 * 1024 * 1024),
    )(xt, s1, b1, w1_eff, b2, w2g)

    return jnp.transpose(out_t.reshape(H, W, N, Cin + Cout), (2, 3, 0, 1))


# speedup vs baseline: 10.4449x; 1.0052x over previous
import functools

import jax
import jax.numpy as jnp
from jax.experimental import pallas as pl
from jax.experimental.pallas import tpu as pltpu


def _dense_fused_kernel(x_ref, s1_ref, b1_ref, w1_ref, b2_ref, w2g_ref,
                        o_ref, *, H, W, B, Cin, C1, Cout):
    HW = H * W
    M = HW * B
    SH = W * B

    o_ref[:, :, :Cin] = x_ref[...]

    x2 = x_ref[...].reshape(M, Cin)

    h = jnp.maximum(x2 * s1_ref[...] + b1_ref[...], 0.0)

    t = jnp.dot(h, w1_ref[...], preferred_element_type=jnp.float32)
    t = jnp.maximum(t + b2_ref[...], 0.0)

    WB = W * B

    def colshift(p, k):
        z = jnp.zeros((H, abs(k), p.shape[2]), jnp.float32)
        if k > 0:
            return jnp.concatenate([p[:, k:], z], axis=1)
        return jnp.concatenate([z, p[:, :WB + k]], axis=1)

    def rowshift(p, k):
        z = jnp.zeros((abs(k), WB, p.shape[2]), jnp.float32)
        if k > 0:
            return jnp.concatenate([p[k:], z], axis=0)
        return jnp.concatenate([z, p[:H + k]], axis=0)

    t3 = t.reshape(H, WB, C1)
    cat = jnp.concatenate([colshift(t3, -B), t3, colshift(t3, B)],
                          axis=2).reshape(M, 3 * C1)
    p = jnp.dot(cat, w2g_ref[...], preferred_element_type=jnp.float32)
    p3 = p.reshape(H, WB, 3 * Cout)

    y = (p3[:, :, Cout:2 * Cout]
         + rowshift(p3[:, :, :Cout], -1)
         + rowshift(p3[:, :, 2 * Cout:], 1))

    o_ref[:, :, Cin:] = y.reshape(HW, B, Cout)


def kernel(x, s1, b1, w1_eff, b2, w2):
    N, Cin, H, W = x.shape
    C1 = w1_eff.shape[1]
    Cout = w2.shape[1]
    HW = H * W
    B = 8 if N % 8 == 0 else N
    G = N // B
    NC = 1
    GK = G // NC

    w2g = jnp.transpose(w2.reshape(3, 3, C1, Cout),
                        (1, 2, 0, 3)).reshape(3 * C1, 3 * Cout)

    xt = jnp.transpose(x, (2, 3, 0, 1)).reshape(HW, N, Cin)

    out_t = pl.pallas_call(
        functools.partial(_dense_fused_kernel, H=H, W=W, B=B, Cin=Cin,
                          C1=C1, Cout=Cout),
        out_shape=jax.ShapeDtypeStruct((HW, N, Cin + Cout), jnp.float32),
        grid=(NC, GK),
        in_specs=[
            pl.BlockSpec((HW, B, Cin), lambda c, k: (0, c * GK + k, 0)),
            pl.BlockSpec((1, Cin), lambda c, k: (0, 0)),
            pl.BlockSpec((1, Cin), lambda c, k: (0, 0)),
            pl.BlockSpec((Cin, C1), lambda c, k: (0, 0)),
            pl.BlockSpec((1, C1), lambda c, k: (0, 0)),
            pl.BlockSpec((3 * C1, 3 * Cout), lambda c, k: (0, 0)),
        ],
        out_specs=pl.BlockSpec((HW, B, Cin + Cout),
                               lambda c, k: (0, c * GK + k, 0)),
        compiler_params=pltpu.CompilerParams(
            dimension_semantics=("arbitrary", "arbitrary"),
            vmem_limit_bytes=100 * 1024 * 1024),
    )(xt, s1, b1, w1_eff, b2, w2g)

    return jnp.transpose(out_t.reshape(H, W, N, Cin + Cout), (2, 3, 0, 1))


# speedup vs baseline: 10.4463x; 1.0001x over previous
import functools

import jax
import jax.numpy as jnp
from jax.experimental import pallas as pl
from jax.experimental.pallas import tpu as pltpu


def _dense_fused_kernel(x_ref, s1_ref, b1_ref, w1_ref, b2_ref, w2g_ref,
                        o_ref, *, H, W, B, Cin, C1, Cout):
    HW = H * W
    M = HW * B
    SH = W * B

    o_ref[:, :, :Cin] = x_ref[...]

    x2 = x_ref[...].reshape(M, Cin)

    h = jnp.maximum(x2 * s1_ref[...] + b1_ref[...], 0.0)

    t = jnp.dot(h, w1_ref[...], preferred_element_type=jnp.float32)
    t = jnp.maximum(t + b2_ref[...], 0.0)

    WB = W * B

    def colshift(p, k):
        z = jnp.zeros((H, abs(k), p.shape[2]), jnp.float32)
        if k > 0:
            return jnp.concatenate([p[:, k:], z], axis=1)
        return jnp.concatenate([z, p[:, :WB + k]], axis=1)

    def rowshift(p, k):
        z = jnp.zeros((abs(k), WB, p.shape[2]), jnp.float32)
        if k > 0:
            return jnp.concatenate([p[k:], z], axis=0)
        return jnp.concatenate([z, p[:H + k]], axis=0)

    t3 = t.reshape(H, WB, C1)
    cat = jnp.concatenate([colshift(t3, -B), t3, colshift(t3, B)],
                          axis=2).reshape(M, 3 * C1)
    p = jnp.dot(cat, w2g_ref[...], preferred_element_type=jnp.float32)
    p3 = p.reshape(H, WB, 3 * Cout)

    y = (p3[:, :, Cout:2 * Cout]
         + rowshift(p3[:, :, :Cout], -1)
         + rowshift(p3[:, :, 2 * Cout:], 1))

    o_ref[:, :, Cin:] = y.reshape(HW, B, Cout)


def kernel(x, s1, b1, w1_eff, b2, w2):
    N, Cin, H, W = x.shape
    C1 = w1_eff.shape[1]
    Cout = w2.shape[1]
    HW = H * W
    B = 8 if N % 8 == 0 else N
    G = N // B

    w2g = jnp.transpose(w2.reshape(3, 3, C1, Cout),
                        (1, 2, 0, 3)).reshape(3 * C1, 3 * Cout)

    xt = jnp.transpose(x, (2, 3, 0, 1)).reshape(HW, N, Cin)

    out_t = pl.pallas_call(
        functools.partial(_dense_fused_kernel, H=H, W=W, B=B, Cin=Cin,
                          C1=C1, Cout=Cout),
        out_shape=jax.ShapeDtypeStruct((HW, N, Cin + Cout), jnp.float32),
        grid=(G,),
        in_specs=[
            pl.BlockSpec((HW, B, Cin), lambda k: (0, k, 0)),
            pl.BlockSpec((1, Cin), lambda k: (0, 0)),
            pl.BlockSpec((1, Cin), lambda k: (0, 0)),
            pl.BlockSpec((Cin, C1), lambda k: (0, 0)),
            pl.BlockSpec((1, C1), lambda k: (0, 0)),
            pl.BlockSpec((3 * C1, 3 * Cout), lambda k: (0, 0)),
        ],
        out_specs=pl.BlockSpec((HW, B, Cin + Cout), lambda k: (0, k, 0)),
        compiler_params=pltpu.CompilerParams(
            dimension_semantics=("arbitrary",),
            vmem_limit_bytes=100 * 1024 * 1024),
    )(xt, s1, b1, w1_eff, b2, w2g)

    return jnp.transpose(out_t.reshape(H, W, N, Cin + Cout), (2, 3, 0, 1))
